# baseline (speedup 1.0000x reference)
import functools

import jax
import jax.numpy as jnp
from jax import lax
from jax.experimental import pallas as pl
from jax.experimental.pallas import tpu as pltpu
from jax.experimental.pallas import tpu_sc as plsc

B = 16384
D = 64
H = 2 * D
NC = 2
NS = 16
NW = NC * NS
BPW = B // NW
L = 16
TB = 2048
NB = B // TB
AVG_R = 3.5
LAM = 0.001


def _sc_gather(user0, item_i0, ratings, embed_user, embed_item,
               user_bias, item_bias):
    mesh = plsc.VectorSubcoreMesh(core_axis_name="c", subcore_axis_name="s")

    @functools.partial(
        pl.kernel,
        mesh=mesh,
        out_type=(
            jax.ShapeDtypeStruct((2 * B, D), jnp.float32),
            jax.ShapeDtypeStruct((B,), jnp.float32),
        ),
        scratch_types=(
            pltpu.VMEM((BPW,), jnp.int32),
            pltpu.VMEM((BPW,), jnp.int32),
            pltpu.VMEM((BPW,), jnp.float32),
            pltpu.VMEM((BPW,), jnp.float32),
            pltpu.VMEM((BPW,), jnp.float32),
            pltpu.VMEM((BPW,), jnp.float32),
            pltpu.SemaphoreType.DMA,
            pltpu.SemaphoreType.DMA,
            pltpu.SemaphoreType.DMA,
            pltpu.SemaphoreType.DMA,
            pltpu.SemaphoreType.DMA,
        ),
    )
    def gather_kernel(u0_hbm, i0_hbm, rat_hbm, eu_hbm, ei_hbm, ubt_hbm,
                      ibt_hbm,
                      rows_out, c_out,
                      uidx_v, iidx_v,
                      ubv, ibv, ratv, cv,
                      sem_u, sem_i, sem_ub, sem_ib, sem_r):
        wid = lax.axis_index("s") * NC + lax.axis_index("c")
        base = pl.multiple_of(wid * BPW, BPW)
        pltpu.sync_copy(u0_hbm.at[pl.ds(base, BPW)], uidx_v)
        pltpu.sync_copy(i0_hbm.at[pl.ds(base, BPW)], iidx_v)
        cub = pltpu.async_copy(ubt_hbm.at[uidx_v], ubv, sem_ub)
        cib = pltpu.async_copy(ibt_hbm.at[iidx_v], ibv, sem_ib)
        crat = pltpu.async_copy(rat_hbm.at[pl.ds(base, BPW)], ratv, sem_r)
        def row_dma(t, carry):
            s = pl.multiple_of(t * L, L)
            uvec = uidx_v[pl.ds(s, L)]
            ivec = iidx_v[pl.ds(s, L)]
            for k in range(L):
                ru = uvec[k]
                ri = ivec[k]
                pltpu.async_copy(eu_hbm.at[pl.ds(ru, 1)],
                                 rows_out.at[pl.ds(base + s + k, 1)],
                                 sem_u)
                pltpu.async_copy(ei_hbm.at[pl.ds(ri, 1)],
                                 rows_out.at[pl.ds(B + base + s + k, 1)],
                                 sem_i)
            return carry

        lax.fori_loop(0, BPW // L, row_dma, 0)

        cub.wait()
        cib.wait()
        crat.wait()

        def c_chunk(k, carry):
            s = pl.multiple_of(k * L, L)
            cv[pl.ds(s, L)] = (ubv[pl.ds(s, L)] + ibv[pl.ds(s, L)]
                               + AVG_R - ratv[pl.ds(s, L)])
            return carry

        lax.fori_loop(0, BPW // L, c_chunk, 0, unroll=4)
        pltpu.sync_copy(cv, c_out.at[pl.ds(base, BPW)])

        pltpu.make_async_copy(eu_hbm.at[pl.ds(0, BPW)],
                              rows_out.at[pl.ds(base, BPW)], sem_u).wait()
        pltpu.make_async_copy(ei_hbm.at[pl.ds(0, BPW)],
                              rows_out.at[pl.ds(B + base, BPW)],
                              sem_i).wait()

    return gather_kernel(user0, item_i0, ratings, embed_user, embed_item,
                         user_bias, item_bias)


def _tc_body(u_ref, it_ref, c_ref,
             W1_ref, b1_ref, W2_ref, b2_ref, part_ref):
    W1 = W1_ref[...]
    b1 = b1_ref[...]
    W2 = W2_ref[...]
    b2 = b2_ref[...]

    def mlp(x):
        h = jnp.dot(x, W1, preferred_element_type=jnp.float32) + b1
        h = jnp.where(h >= 0, h, 0.1 * h)
        return jnp.dot(h, W2, preferred_element_type=jnp.float32) + b2

    fu = mlp(u_ref[...])
    fi = mlp(it_ref[...])
    dots = jnp.sum(fu * fi, axis=1)
    err = dots + c_ref[...]
    i = pl.program_id(0)
    part_ref[i, 0] = jnp.sum(err * err)
    part_ref[i, 1] = jnp.sum(fu * fu)
    part_ref[i, 2] = jnp.sum(fi * fi)


def _tc_loss(rows, c, W1, b1, W2, b2):
    return pl.pallas_call(
        _tc_body,
        grid=(NB,),
        in_specs=[
            pl.BlockSpec((TB, D), lambda i: (i, 0)),
            pl.BlockSpec((TB, D), lambda i: (NB + i, 0)),
            pl.BlockSpec((TB,), lambda i: (i,)),
            pl.BlockSpec((D, H), lambda i: (0, 0)),
            pl.BlockSpec((1, H), lambda i: (0, 0)),
            pl.BlockSpec((H, D), lambda i: (0, 0)),
            pl.BlockSpec((1, D), lambda i: (0, 0)),
        ],
        out_specs=pl.BlockSpec(memory_space=pltpu.SMEM),
        out_shape=jax.ShapeDtypeStruct((NB, 3), jnp.float32),
    )(rows, rows, c, W1, b1, W2, b2)


def kernel(user0, item_i0, ratings, embed_user, embed_item,
           W1, b1, W2, b2, user_bias, item_bias):
    rows, c = _sc_gather(
        user0.astype(jnp.int32), item_i0.astype(jnp.int32),
        ratings.astype(jnp.float32),
        embed_user, embed_item, user_bias[:, 0], item_bias[:, 0])
    parts = _tc_loss(rows, c, W1, b1.reshape(1, H), W2, b2.reshape(1, D))
    sums = jnp.sum(parts, axis=0)
    loss2 = sums[0] / B
    l2 = LAM * (sums[1] / (B * D)) + LAM * (sums[2] / (B * D))
    loss = loss2 + l2
    z = jnp.float32(0.0)
    return (loss, loss2, z, z, z, z)


# speedup vs baseline: 2.8129x; 2.8129x over previous
import functools

import jax
import jax.numpy as jnp
from jax import lax
from jax.experimental import pallas as pl
from jax.experimental.pallas import tpu as pltpu
from jax.experimental.pallas import tpu_sc as plsc

B = 16384
D = 64
H = 2 * D
V = 100000
NC = 2
NS = 16
NW = NC * NS
BPW = B // NW
CH = BPW // 2
L = 16
TB = 2048
NB = B // TB
AVG_R = 3.5
LAM = 0.001


def _sc_gather(user0, item_i0, ratings, eu2, ei2, user_bias, item_bias):
    mesh = plsc.VectorSubcoreMesh(core_axis_name="c", subcore_axis_name="s")

    @functools.partial(
        pl.kernel,
        mesh=mesh,
        out_type=(
            jax.ShapeDtypeStruct((2 * B, 128), jnp.float32),
            jax.ShapeDtypeStruct((B,), jnp.float32),
        ),
        scratch_types=(
            pltpu.VMEM((BPW,), jnp.int32),
            pltpu.VMEM((BPW,), jnp.int32),
            pltpu.VMEM((BPW,), jnp.int32),
            pltpu.VMEM((BPW,), jnp.int32),
            pltpu.VMEM((CH, 128), jnp.float32),
            pltpu.VMEM((CH, 128), jnp.float32),
            pltpu.VMEM((BPW,), jnp.float32),
            pltpu.VMEM((BPW,), jnp.float32),
            pltpu.VMEM((BPW,), jnp.float32),
            pltpu.VMEM((BPW,), jnp.float32),
            pltpu.SemaphoreType.DMA,
            pltpu.SemaphoreType.DMA,
            pltpu.SemaphoreType.DMA,
            pltpu.SemaphoreType.DMA,
            pltpu.SemaphoreType.DMA,
        ),
    )
    def gather_kernel(u0_hbm, i0_hbm, rat_hbm, eu_hbm, ei_hbm, ubt_hbm,
                      ibt_hbm,
                      rows_out, c_out,
                      uidx_v, iidx_v, upair_v, ipair_v,
                      ubuf, ibuf, ubv, ibv, ratv, cv,
                      sem_u, sem_i, sem_ub, sem_ib, sem_r):
        wid = lax.axis_index("s") * NC + lax.axis_index("c")
        base = pl.multiple_of(wid * BPW, BPW)
        pltpu.sync_copy(u0_hbm.at[pl.ds(base, BPW)], uidx_v)
        pltpu.sync_copy(i0_hbm.at[pl.ds(base, BPW)], iidx_v)
        cub = pltpu.async_copy(ubt_hbm.at[uidx_v], ubv, sem_ub)
        cib = pltpu.async_copy(ibt_hbm.at[iidx_v], ibv, sem_ib)
        crat = pltpu.async_copy(rat_hbm.at[pl.ds(base, BPW)], ratv, sem_r)

        def pair_chunk(k, carry):
            s = pl.multiple_of(k * L, L)
            upair_v[pl.ds(s, L)] = lax.shift_right_logical(
                uidx_v[pl.ds(s, L)], 1)
            ipair_v[pl.ds(s, L)] = lax.shift_right_logical(
                iidx_v[pl.ds(s, L)], 1)
            return carry

        lax.fori_loop(0, BPW // L, pair_chunk, 0, unroll=4)

        cu0 = pltpu.async_copy(eu_hbm.at[upair_v.at[pl.ds(0, CH)]],
                               ubuf, sem_u)
        ci0 = pltpu.async_copy(ei_hbm.at[ipair_v.at[pl.ds(0, CH)]],
                               ibuf, sem_i)
        cu0.wait()
        pltpu.sync_copy(ubuf, rows_out.at[pl.ds(base, CH)])
        cu1 = pltpu.async_copy(eu_hbm.at[upair_v.at[pl.ds(CH, CH)]],
                               ubuf, sem_u)
        ci0.wait()
        pltpu.sync_copy(ibuf, rows_out.at[pl.ds(B + base, CH)])
        ci1 = pltpu.async_copy(ei_hbm.at[ipair_v.at[pl.ds(CH, CH)]],
                               ibuf, sem_i)
        cub.wait()
        cib.wait()
        crat.wait()

        def c_chunk(k, carry):
            s = pl.multiple_of(k * L, L)
            cv[pl.ds(s, L)] = (ubv[pl.ds(s, L)] + ibv[pl.ds(s, L)]
                               + AVG_R - ratv[pl.ds(s, L)])
            return carry

        lax.fori_loop(0, BPW // L, c_chunk, 0, unroll=4)
        pltpu.sync_copy(cv, c_out.at[pl.ds(base, BPW)])

        cu1.wait()
        pltpu.sync_copy(ubuf, rows_out.at[pl.ds(base + CH, CH)])
        ci1.wait()
        pltpu.sync_copy(ibuf, rows_out.at[pl.ds(B + base + CH, CH)])

    return gather_kernel(user0, item_i0, ratings, eu2, ei2,
                         user_bias, item_bias)


def _tc_body(u_ref, it_ref, u0_ref, i0_ref, c_ref,
             W1_ref, b1_ref, W2_ref, b2_ref, part_ref):
    W1 = W1_ref[...]
    b1 = b1_ref[...]
    W2 = W2_ref[...]
    b2 = b2_ref[...]

    def mlp(x):
        h = jnp.dot(x, W1, preferred_element_type=jnp.float32) + b1
        h = jnp.where(h >= 0, h, 0.1 * h)
        return jnp.dot(h, W2, preferred_element_type=jnp.float32) + b2

    xu2 = u_ref[...]
    xi2 = it_ref[...]
    pu = (u0_ref[...] & 1) == 0
    pi = (i0_ref[...] & 1) == 0
    fu_l = mlp(xu2[:, :D])
    fu_r = mlp(xu2[:, D:])
    fi_l = mlp(xi2[:, :D])
    fi_r = mlp(xi2[:, D:])
    dll = jnp.sum(fu_l * fi_l, axis=1)
    dlr = jnp.sum(fu_l * fi_r, axis=1)
    drl = jnp.sum(fu_r * fi_l, axis=1)
    drr = jnp.sum(fu_r * fi_r, axis=1)
    dots = jnp.where(pu, jnp.where(pi, dll, dlr), jnp.where(pi, drl, drr))
    err = dots + c_ref[...]
    squ = jnp.where(pu, jnp.sum(fu_l * fu_l, axis=1),
                    jnp.sum(fu_r * fu_r, axis=1))
    sqi = jnp.where(pi, jnp.sum(fi_l * fi_l, axis=1),
                    jnp.sum(fi_r * fi_r, axis=1))
    i = pl.program_id(0)
    part_ref[i, 0] = jnp.sum(err * err)
    part_ref[i, 1] = jnp.sum(squ)
    part_ref[i, 2] = jnp.sum(sqi)


def _tc_loss(rows2, u0, i0, c, W1, b1, W2, b2):
    return pl.pallas_call(
        _tc_body,
        grid=(NB,),
        in_specs=[
            pl.BlockSpec((TB, 128), lambda i: (i, 0)),
            pl.BlockSpec((TB, 128), lambda i: (NB + i, 0)),
            pl.BlockSpec((TB,), lambda i: (i,)),
            pl.BlockSpec((TB,), lambda i: (i,)),
            pl.BlockSpec((TB,), lambda i: (i,)),
            pl.BlockSpec((D, H), lambda i: (0, 0)),
            pl.BlockSpec((1, H), lambda i: (0, 0)),
            pl.BlockSpec((H, D), lambda i: (0, 0)),
            pl.BlockSpec((1, D), lambda i: (0, 0)),
        ],
        out_specs=pl.BlockSpec(memory_space=pltpu.SMEM),
        out_shape=jax.ShapeDtypeStruct((NB, 3), jnp.float32),
    )(rows2, rows2, u0, i0, c, W1, b1, W2, b2)


def kernel(user0, item_i0, ratings, embed_user, embed_item,
           W1, b1, W2, b2, user_bias, item_bias):
    u0 = user0.astype(jnp.int32)
    i0 = item_i0.astype(jnp.int32)
    eu2 = embed_user.reshape(V // 2, 2 * D)
    ei2 = embed_item.reshape(V // 2, 2 * D)
    rows2, c = _sc_gather(u0, i0, ratings.astype(jnp.float32), eu2, ei2,
                          user_bias[:, 0], item_bias[:, 0])
    parts = _tc_loss(rows2, u0, i0, c,
                     W1, b1.reshape(1, H), W2, b2.reshape(1, D))
    sums = jnp.sum(parts, axis=0)
    loss2 = sums[0] / B
    l2 = LAM * (sums[1] / (B * D)) + LAM * (sums[2] / (B * D))
    loss = loss2 + l2
    z = jnp.float32(0.0)
    return (loss, loss2, z, z, z, z)
